# baseline (speedup 1.0000x reference)
import functools

import jax
import jax.numpy as jnp
from jax import lax
from jax.experimental import pallas as pl
from jax.experimental.pallas import tpu as pltpu
from jax.experimental.pallas import tpu_sc as plsc

D = 1024
H = 2048
E = 8
M = 4096
TILE = 128
NT = 40
MPAD = NT * TILE
RT = 512
NRT = M // RT

NW = 32
TPW = M // NW
G = 16
NG = TPW // G


def _router_body(x_ref, wr_ref, br_ref, probs_ref, eidx_ref, pos_ref,
                 ord_ref, ebo_ref, first_ref, nact_ref, nord_ref,
                 counts_ref, eidx_s, rank_s):
    i = pl.program_id(0)

    @pl.when(i == 0)
    def _init():
        counts_ref[...] = jnp.zeros_like(counts_ref)

    logits = jnp.dot(x_ref[...], wr_ref[...]) + br_ref[...]
    probs = jax.nn.softmax(logits, axis=-1)
    probs_ref[...] = probs

    mx = jnp.max(probs, axis=1, keepdims=True)
    e_iota = lax.broadcasted_iota(jnp.int32, (RT, E), 1)
    eidx = jnp.min(jnp.where(probs == mx, e_iota, E), axis=1, keepdims=True)
    eidx_s[pl.ds(i * RT, RT), :] = eidx

    onehot = (e_iota == eidx).astype(jnp.float32)
    r_iota = lax.broadcasted_iota(jnp.int32, (RT, RT), 0)
    c_iota = lax.broadcasted_iota(jnp.int32, (RT, RT), 1)
    tril = (c_iota < r_iota).astype(jnp.float32)
    ranks_excl = jnp.dot(tril, onehot)
    rank = jnp.sum(onehot * (ranks_excl + counts_ref[...]), axis=1,
                   keepdims=True)
    rank_s[pl.ds(i * RT, RT), :] = rank.astype(jnp.int32)
    counts_ref[...] += jnp.sum(onehot, axis=0, keepdims=True)

    @pl.when(i == NRT - 1)
    def _finalize():
        c = counts_ref[...]
        pc = jnp.floor((c + (TILE - 1)) * (1.0 / TILE)) * TILE
        ee_r = lax.broadcasted_iota(jnp.int32, (E, E), 0)
        ee_c = lax.broadcasted_iota(jnp.int32, (E, E), 1)
        i8 = (ee_r == ee_c).astype(jnp.float32)
        excl = (ee_r < ee_c).astype(jnp.float32)
        off_excl = jnp.dot(pc, excl)
        dn = (((0,), (0,)), ((), ()))
        pc_c = lax.dot_general(i8, pc, (((1,), (1,)), ((), ())))
        low = (ee_c < ee_r).astype(jnp.float32)
        off_c = jnp.dot(low, pc_c)
        offi_c = off_c + pc_c
        offd_c = off_c * (1.0 / TILE)
        m_c = (pc_c > 0.0).astype(jnp.float32)
        ord_c = jnp.dot(low, m_c)
        nord_ref[...] = jnp.sum(m_c, axis=0, keepdims=True).astype(jnp.int32)
        total = jnp.sum(pc, axis=1, keepdims=True)
        nact_ref[...] = (total * (1.0 / TILE)).astype(jnp.int32)

        t_iota = lax.broadcasted_iota(jnp.int32, (E, 64), 1).astype(
            jnp.float32)
        texp = jnp.sum((t_iota * TILE >= offi_c).astype(jnp.float32),
                       axis=0, keepdims=True)
        texp = jnp.minimum(texp, float(E - 1))
        first_ref[...] = jnp.sum(
            (t_iota == offd_c).astype(jnp.float32) * m_c, axis=0,
            keepdims=True).astype(jnp.int32)
        e_iota64 = lax.broadcasted_iota(jnp.int32, (E, 64), 0).astype(
            jnp.float32)
        ord_ref[...] = jnp.sum(
            (texp == e_iota64).astype(jnp.float32) * ord_c, axis=0,
            keepdims=True).astype(jnp.int32)
        k_iota = lax.broadcasted_iota(jnp.int32, (E, E), 1).astype(
            jnp.float32)
        e_iota8 = lax.broadcasted_iota(jnp.int32, (E, E), 0).astype(
            jnp.float32)
        ebo_ref[...] = jnp.sum(
            (ord_c == k_iota).astype(jnp.float32) * m_c * e_iota8, axis=0,
            keepdims=True).astype(jnp.int32)
        ef = eidx_s[...].astype(jnp.float32)
        e_iota_m = lax.broadcasted_iota(jnp.int32, (M, E), 1).astype(
            jnp.float32)
        oh_m = (ef == e_iota_m).astype(jnp.float32)
        offs = jnp.sum(oh_m * off_excl, axis=1, keepdims=True)
        pos_col = offs + rank_s[...].astype(jnp.float32)
        rr = lax.broadcasted_iota(jnp.int32, (RT, RT), 0)
        rc = lax.broadcasted_iota(jnp.int32, (RT, RT), 1)
        irt = (rr == rc).astype(jnp.float32)
        for t in range(NRT):
            lo, hi = t * RT, (t + 1) * RT
            pos_ref[:, lo:hi] = lax.dot_general(
                pos_col[lo:hi, :], irt, dn,
                precision=lax.Precision.HIGHEST).astype(jnp.int32)
            eidx_ref[:, lo:hi] = lax.dot_general(
                ef[lo:hi, :], irt, dn,
                precision=lax.Precision.HIGHEST).astype(jnp.int32)


def _router(xf, Wr, br2):
    return pl.pallas_call(
        _router_body,
        grid=(NRT,),
        in_specs=[
            pl.BlockSpec((RT, D), lambda i: (i, 0)),
            pl.BlockSpec((D, E), lambda i: (0, 0)),
            pl.BlockSpec((1, E), lambda i: (0, 0)),
        ],
        out_specs=[
            pl.BlockSpec((RT, E), lambda i: (i, 0)),
            pl.BlockSpec((1, M), lambda i: (0, 0)),
            pl.BlockSpec((1, M), lambda i: (0, 0)),
            pl.BlockSpec((1, 64), lambda i: (0, 0)),
            pl.BlockSpec((1, E), lambda i: (0, 0)),
            pl.BlockSpec((1, 64), lambda i: (0, 0)),
            pl.BlockSpec((1, 1), lambda i: (0, 0)),
            pl.BlockSpec((1, 1), lambda i: (0, 0)),
        ],
        out_shape=[
            jax.ShapeDtypeStruct((M, E), jnp.float32),
            jax.ShapeDtypeStruct((1, M), jnp.int32),
            jax.ShapeDtypeStruct((1, M), jnp.int32),
            jax.ShapeDtypeStruct((1, 64), jnp.int32),
            jax.ShapeDtypeStruct((1, E), jnp.int32),
            jax.ShapeDtypeStruct((1, 64), jnp.int32),
            jax.ShapeDtypeStruct((1, 1), jnp.int32),
            jax.ShapeDtypeStruct((1, 1), jnp.int32),
        ],
        scratch_shapes=[
            pltpu.VMEM((1, E), jnp.float32),
            pltpu.VMEM((M, 1), jnp.int32),
            pltpu.VMEM((M, 1), jnp.int32),
        ],
    )(xf, Wr, br2)


def _dispatch_body(x_hbm, pos_hbm, xs_hbm, pos_v, rows_v, isem, osem):
    wid = lax.axis_index("s") * 2 + lax.axis_index("c")
    base = wid * TPW
    pltpu.sync_copy(pos_hbm.at[pl.ds(base, TPW)], pos_v)
    ins = [pltpu.async_copy(x_hbm.at[pl.ds(base + g * G, G)], rows_v.at[g],
                            isem) for g in range(3)]
    outs = []
    for g in range(NG):
        ins[g].wait()
        pos16 = pos_v[pl.ds(g * G, G)]
        outs.append(pltpu.async_copy(rows_v.at[g % 4], xs_hbm.at[pos16],
                                     osem))
        if g >= 1:
            outs[g - 1].wait()
        if g + 3 < NG:
            ins.append(pltpu.async_copy(
                x_hbm.at[pl.ds(base + (g + 3) * G, G)],
                rows_v.at[(g + 3) % 4], isem))
    outs[NG - 1].wait()


def _combine_body(ys_hbm, pos_hbm, out_hbm, pos_v, rows_v, isem, osem):
    wid = lax.axis_index("s") * 2 + lax.axis_index("c")
    base = wid * TPW
    b = wid // (NW // 2)
    brow = (wid % (NW // 2)) * TPW
    pltpu.sync_copy(pos_hbm.at[pl.ds(base, TPW)], pos_v)
    ins = [pltpu.async_copy(ys_hbm.at[pos_v[pl.ds(g * G, G)]], rows_v.at[g],
                            isem) for g in range(3)]
    outs = []
    for g in range(NG):
        ins[g].wait()
        outs.append(pltpu.async_copy(
            rows_v.at[g % 4], out_hbm.at[b, pl.ds(brow + g * G, G)], osem))
        if g >= 1:
            outs[g - 1].wait()
        if g + 3 < NG:
            pos16 = pos_v[pl.ds((g + 3) * G, G)]
            ins.append(pltpu.async_copy(ys_hbm.at[pos16],
                                        rows_v.at[(g + 3) % 4], isem))
    outs[NG - 1].wait()


@functools.cache
def _sc_kernels():
    mesh = plsc.VectorSubcoreMesh(core_axis_name="c", subcore_axis_name="s")
    dispatch = functools.partial(
        pl.kernel,
        out_type=jax.ShapeDtypeStruct((MPAD, D), jnp.float32),
        mesh=mesh,
        scratch_types=[
            pltpu.VMEM((TPW,), jnp.int32),
            pltpu.VMEM((4, G, D), jnp.float32),
            pltpu.SemaphoreType.DMA,
            pltpu.SemaphoreType.DMA,
        ],
    )(_dispatch_body)
    combine = functools.partial(
        pl.kernel,
        out_type=jax.ShapeDtypeStruct((2, M // 2, D), jnp.float32),
        mesh=mesh,
        scratch_types=[
            pltpu.VMEM((TPW,), jnp.int32),
            pltpu.VMEM((4, G, D), jnp.float32),
            pltpu.SemaphoreType.DMA,
            pltpu.SemaphoreType.DMA,
        ],
    )(_combine_body)
    return dispatch, combine


_WCHUNKS = 4
_WSLOTS = 3


def _weight_copies(w1_hbm, w2_hbm, w1buf, w2buf, sems, q, s):
    c1, c2 = D // _WCHUNKS, H // _WCHUNKS
    descs = []
    for c in range(_WCHUNKS):
        descs.append(pltpu.make_async_copy(
            w1_hbm.at[q, pl.ds(c * c1, c1)], w1buf.at[s, pl.ds(c * c1, c1)],
            sems.at[s]))
        descs.append(pltpu.make_async_copy(
            w2_hbm.at[q, pl.ds(c * c2, c2)], w2buf.at[s, pl.ds(c * c2, c2)],
            sems.at[s]))
    return descs


def _gmm_body(ord_sm, ebo_sm, first_sm, nact_sm, nord_sm,
              xs_ref, w1_hbm, w2_hbm, b1_ref, b2_ref, ys_ref,
              w1buf, w2buf, sems):
    i = pl.program_id(0)
    k = ord_sm[i]
    slot = lax.rem(k, _WSLOTS)

    @pl.when(i == 0)
    def _prologue():
        for kk in range(_WSLOTS):
            @pl.when(kk < nord_sm[0])
            def _issue():
                for d in _weight_copies(w1_hbm, w2_hbm, w1buf, w2buf, sems,
                                        ebo_sm[kk], kk):
                    d.start()

    @pl.when((i > 0) & (first_sm[i] == 1)
             & (k + (_WSLOTS - 1) < nord_sm[0]))
    def _advance():
        kn = k + (_WSLOTS - 1)
        for d in _weight_copies(w1_hbm, w2_hbm, w1buf, w2buf, sems,
                                ebo_sm[kn], lax.rem(kn, _WSLOTS)):
            d.start()

    @pl.when(first_sm[i] == 1)
    def _await_weights():
        for d in _weight_copies(w1_hbm, w2_hbm, w1buf, w2buf, sems,
                                ebo_sm[k], slot):
            d.wait()

    @pl.when(i < nact_sm[0])
    def _compute():
        q = ebo_sm[k]
        h = jnp.maximum(jnp.dot(xs_ref[...], w1buf[slot]) + b1_ref[q], 0.0)
        ys_ref[...] = jnp.dot(h, w2buf[slot]) + b2_ref[q]


def _gmm(ord_t, ebo, first, nact, nord, xs, W1, b1, W2, b2):
    grid_spec = pltpu.PrefetchScalarGridSpec(
        num_scalar_prefetch=5,
        grid=(NT,),
        in_specs=[
            pl.BlockSpec((TILE, D), lambda i, *_: (i, 0)),
            pl.BlockSpec(memory_space=pltpu.MemorySpace.HBM),
            pl.BlockSpec(memory_space=pltpu.MemorySpace.HBM),
            pl.BlockSpec((E, 1, H), lambda i, *_: (0, 0, 0)),
            pl.BlockSpec((E, 1, D), lambda i, *_: (0, 0, 0)),
        ],
        out_specs=pl.BlockSpec((TILE, D), lambda i, *_: (i, 0)),
        scratch_shapes=[
            pltpu.VMEM((_WSLOTS, D, H), jnp.float32),
            pltpu.VMEM((_WSLOTS, H, D), jnp.float32),
            pltpu.SemaphoreType.DMA((_WSLOTS,)),
        ],
    )
    return pl.pallas_call(
        _gmm_body,
        grid_spec=grid_spec,
        out_shape=jax.ShapeDtypeStruct((MPAD, D), jnp.float32),
    )(ord_t, ebo, first, nact, nord, xs, W1, W2, b1.reshape(E, 1, H),
      b2.reshape(E, 1, D))


def kernel(x, Wr, br, W1, b1, W2, b2):
    B, T, _ = x.shape
    xf = x.reshape(M, D)
    (probs, eidx2, pos2, ord2, ebo2, first2, nact2, nord2) = _router(
        xf, Wr, br.reshape(1, E))
    pos = pos2.reshape(M)
    dispatch, combine = _sc_kernels()
    xs = dispatch(xf, pos)
    ys = _gmm(ord2.reshape(64), ebo2.reshape(E), first2.reshape(64),
              nact2.reshape(1), nord2.reshape(1), xs, W1, b1, W2, b2)
    out = combine(ys, pos)
    return (out, probs.reshape(B, T, E), eidx2.reshape(B, T))


# speedup vs baseline: 1.0145x; 1.0145x over previous
import functools

import jax
import jax.numpy as jnp
from jax import lax
from jax.experimental import pallas as pl
from jax.experimental.pallas import tpu as pltpu
from jax.experimental.pallas import tpu_sc as plsc

D = 1024
H = 2048
E = 8
M = 4096
TILE = 128
NT = 40
MPAD = NT * TILE
RT = 512
NRT = M // RT

NW = 32
TPW = M // NW
G = 16
NG = TPW // G


def _router_body(x_ref, wr_ref, br_ref, probs_ref, eidx_ref,
                 pos_ref, ord_ref, ebo_ref, first_ref, nact_ref, nord_ref,
                 counts_ref, eidx_s, rank_s):
    i = pl.program_id(0)

    @pl.when(i == 0)
    def _init():
        counts_ref[...] = jnp.zeros_like(counts_ref)

    logits = jnp.dot(x_ref[...], wr_ref[...]) + br_ref[...]
    probs = jax.nn.softmax(logits, axis=-1)
    probs_ref[...] = probs

    mx = jnp.max(probs, axis=1, keepdims=True)
    e_iota = lax.broadcasted_iota(jnp.int32, (RT, E), 1)
    eidx = jnp.min(jnp.where(probs == mx, e_iota, E), axis=1, keepdims=True)
    eidx_s[pl.ds(i * RT, RT), :] = eidx

    onehot = (e_iota == eidx).astype(jnp.float32)
    r_iota = lax.broadcasted_iota(jnp.int32, (RT, RT), 0)
    c_iota = lax.broadcasted_iota(jnp.int32, (RT, RT), 1)
    tril = (c_iota < r_iota).astype(jnp.float32)
    ranks_excl = jnp.dot(tril, onehot)
    rank = jnp.sum(onehot * (ranks_excl + counts_ref[...]), axis=1,
                   keepdims=True)
    rank_s[pl.ds(i * RT, RT), :] = rank.astype(jnp.int32)
    counts_ref[...] += jnp.sum(onehot, axis=0, keepdims=True)

    @pl.when(i == NRT - 1)
    def _finalize():
        c = counts_ref[...]
        pc = jnp.floor((c + (TILE - 1)) * (1.0 / TILE)) * TILE
        ee_r = lax.broadcasted_iota(jnp.int32, (E, E), 0)
        ee_c = lax.broadcasted_iota(jnp.int32, (E, E), 1)
        i8 = (ee_r == ee_c).astype(jnp.float32)
        excl = (ee_r < ee_c).astype(jnp.float32)
        off_excl = jnp.dot(pc, excl)
        dn = (((0,), (0,)), ((), ()))
        pc_c = lax.dot_general(i8, pc, (((1,), (1,)), ((), ())))
        low = (ee_c < ee_r).astype(jnp.float32)
        off_c = jnp.dot(low, pc_c)
        offi_c = off_c + pc_c
        offd_c = off_c * (1.0 / TILE)
        m_c = (pc_c > 0.0).astype(jnp.float32)
        ord_c = jnp.dot(low, m_c)
        nord_ref[...] = jnp.sum(m_c, axis=0, keepdims=True).astype(jnp.int32)
        total = jnp.sum(pc, axis=1, keepdims=True)
        nact_ref[...] = (total * (1.0 / TILE)).astype(jnp.int32)

        t_iota = lax.broadcasted_iota(jnp.int32, (E, 64), 1).astype(
            jnp.float32)
        texp = jnp.sum((t_iota * TILE >= offi_c).astype(jnp.float32),
                       axis=0, keepdims=True)
        texp = jnp.minimum(texp, float(E - 1))
        first_ref[...] = jnp.sum(
            (t_iota == offd_c).astype(jnp.float32) * m_c, axis=0,
            keepdims=True).astype(jnp.int32)
        e_iota64 = lax.broadcasted_iota(jnp.int32, (E, 64), 0).astype(
            jnp.float32)
        ord_ref[...] = jnp.sum(
            (texp == e_iota64).astype(jnp.float32) * ord_c, axis=0,
            keepdims=True).astype(jnp.int32)
        k_iota = lax.broadcasted_iota(jnp.int32, (E, E), 1).astype(
            jnp.float32)
        e_iota8 = lax.broadcasted_iota(jnp.int32, (E, E), 0).astype(
            jnp.float32)
        ebo_ref[...] = jnp.sum(
            (ord_c == k_iota).astype(jnp.float32) * m_c * e_iota8, axis=0,
            keepdims=True).astype(jnp.int32)
        ef = eidx_s[...].astype(jnp.float32)
        e_iota_m = lax.broadcasted_iota(jnp.int32, (M, E), 1).astype(
            jnp.float32)
        oh_m = (ef == e_iota_m).astype(jnp.float32)
        offs = jnp.sum(oh_m * off_excl, axis=1, keepdims=True)
        pos_col = offs + rank_s[...].astype(jnp.float32)
        rr = lax.broadcasted_iota(jnp.int32, (RT, RT), 0)
        rc = lax.broadcasted_iota(jnp.int32, (RT, RT), 1)
        irt = (rr == rc).astype(jnp.float32)
        for t in range(NRT):
            lo, hi = t * RT, (t + 1) * RT
            pos_ref[:, lo:hi] = lax.dot_general(
                pos_col[lo:hi, :], irt, dn,
                precision=lax.Precision.HIGHEST).astype(jnp.int32)
            eidx_ref[:, lo:hi] = lax.dot_general(
                ef[lo:hi, :], irt, dn,
                precision=lax.Precision.HIGHEST).astype(jnp.int32)


def _router(xf, Wr, br2):
    return pl.pallas_call(
        _router_body,
        grid=(NRT,),
        in_specs=[
            pl.BlockSpec((RT, D), lambda i: (i, 0)),
            pl.BlockSpec((D, E), lambda i: (0, 0)),
            pl.BlockSpec((1, E), lambda i: (0, 0)),
        ],
        out_specs=[
            pl.BlockSpec((RT, E), lambda i: (i, 0)),
            pl.BlockSpec((1, M), lambda i: (0, 0)),
            pl.BlockSpec((1, M), lambda i: (0, 0)),
            pl.BlockSpec((1, 64), lambda i: (0, 0)),
            pl.BlockSpec((1, E), lambda i: (0, 0)),
            pl.BlockSpec((1, 64), lambda i: (0, 0)),
            pl.BlockSpec((1, 1), lambda i: (0, 0)),
            pl.BlockSpec((1, 1), lambda i: (0, 0)),
        ],
        out_shape=[
            jax.ShapeDtypeStruct((M, E), jnp.float32),
            jax.ShapeDtypeStruct((1, M), jnp.int32),
            jax.ShapeDtypeStruct((1, M), jnp.int32),
            jax.ShapeDtypeStruct((1, 64), jnp.int32),
            jax.ShapeDtypeStruct((1, E), jnp.int32),
            jax.ShapeDtypeStruct((1, 64), jnp.int32),
            jax.ShapeDtypeStruct((1, 1), jnp.int32),
            jax.ShapeDtypeStruct((1, 1), jnp.int32),
        ],
        scratch_shapes=[
            pltpu.VMEM((1, E), jnp.float32),
            pltpu.VMEM((M, 1), jnp.int32),
            pltpu.VMEM((M, 1), jnp.int32),
        ],
    )(xf, Wr, br2)


def _dispatch_body(x_hbm, pos_hbm, xs_hbm, pos_v, rows_v, isem, osem):
    wid = lax.axis_index("s") * 2 + lax.axis_index("c")
    base = wid * TPW
    pltpu.sync_copy(pos_hbm.at[pl.ds(base, TPW)], pos_v)
    ins = [pltpu.async_copy(x_hbm.at[pl.ds(base + g * G, G)], rows_v.at[g],
                            isem) for g in range(3)]
    outs = []
    for g in range(NG):
        ins[g].wait()
        pos16 = pos_v[pl.ds(g * G, G)]
        outs.append(pltpu.async_copy(rows_v.at[g % 4], xs_hbm.at[pos16],
                                     osem))
        if g >= 1:
            outs[g - 1].wait()
        if g + 3 < NG:
            ins.append(pltpu.async_copy(
                x_hbm.at[pl.ds(base + (g + 3) * G, G)],
                rows_v.at[(g + 3) % 4], isem))
    outs[NG - 1].wait()


def _combine_body(ys_hbm, pos_hbm, out_hbm, pos_v, rows_v, isem, osem):
    wid = lax.axis_index("s") * 2 + lax.axis_index("c")
    base = wid * TPW
    b = wid // (NW // 2)
    brow = (wid % (NW // 2)) * TPW
    pltpu.sync_copy(pos_hbm.at[pl.ds(base, TPW)], pos_v)
    ins = [pltpu.async_copy(ys_hbm.at[pos_v[pl.ds(g * G, G)]], rows_v.at[g],
                            isem) for g in range(3)]
    outs = []
    for g in range(NG):
        ins[g].wait()
        outs.append(pltpu.async_copy(
            rows_v.at[g % 4], out_hbm.at[b, pl.ds(brow + g * G, G)], osem))
        if g >= 1:
            outs[g - 1].wait()
        if g + 3 < NG:
            pos16 = pos_v[pl.ds((g + 3) * G, G)]
            ins.append(pltpu.async_copy(ys_hbm.at[pos16],
                                        rows_v.at[(g + 3) % 4], isem))
    outs[NG - 1].wait()


@functools.cache
def _sc_kernels():
    mesh = plsc.VectorSubcoreMesh(core_axis_name="c", subcore_axis_name="s")
    dispatch = functools.partial(
        pl.kernel,
        out_type=jax.ShapeDtypeStruct((MPAD, D), jnp.float32),
        mesh=mesh,
        scratch_types=[
            pltpu.VMEM((TPW,), jnp.int32),
            pltpu.VMEM((4, G, D), jnp.float32),
            pltpu.SemaphoreType.DMA,
            pltpu.SemaphoreType.DMA,
        ],
    )(_dispatch_body)
    combine = functools.partial(
        pl.kernel,
        out_type=jax.ShapeDtypeStruct((2, M // 2, D), jnp.float32),
        mesh=mesh,
        scratch_types=[
            pltpu.VMEM((TPW,), jnp.int32),
            pltpu.VMEM((4, G, D), jnp.float32),
            pltpu.SemaphoreType.DMA,
            pltpu.SemaphoreType.DMA,
        ],
    )(_combine_body)
    return dispatch, combine


_WCHUNKS = 4
_WSLOTS = 2


def _weight_copies(w1_hbm, w2_hbm, w1buf, w2buf, sems, q, s):
    c1, c2 = D // _WCHUNKS, H // _WCHUNKS
    descs = []
    for c in range(_WCHUNKS):
        descs.append(pltpu.make_async_copy(
            w1_hbm.at[q, pl.ds(c * c1, c1)], w1buf.at[s, pl.ds(c * c1, c1)],
            sems.at[s]))
        descs.append(pltpu.make_async_copy(
            w2_hbm.at[q, pl.ds(c * c2, c2)], w2buf.at[s, pl.ds(c * c2, c2)],
            sems.at[s]))
    return descs


def _gmm_body(ord_sm, ebo_sm, first_sm, nact_sm, nord_sm,
              xs_ref, w1_hbm, w2_hbm, b1_ref, b2_ref, ys_ref,
              w1buf, w2buf, sems):
    i = pl.program_id(0)
    k = ord_sm[i]
    slot = lax.rem(k, _WSLOTS)

    @pl.when(i == 0)
    def _prologue():
        for kk in range(_WSLOTS):
            @pl.when(kk < nord_sm[0])
            def _issue():
                for d in _weight_copies(w1_hbm, w2_hbm, w1buf, w2buf, sems,
                                        ebo_sm[kk], kk):
                    d.start()

    @pl.when((i > 0) & (first_sm[i] == 1)
             & (k + (_WSLOTS - 1) < nord_sm[0]))
    def _advance():
        kn = k + (_WSLOTS - 1)
        for d in _weight_copies(w1_hbm, w2_hbm, w1buf, w2buf, sems,
                                ebo_sm[kn], lax.rem(kn, _WSLOTS)):
            d.start()

    @pl.when(first_sm[i] == 1)
    def _await_weights():
        for d in _weight_copies(w1_hbm, w2_hbm, w1buf, w2buf, sems,
                                ebo_sm[k], slot):
            d.wait()

    @pl.when(i < nact_sm[0])
    def _compute():
        q = ebo_sm[k]
        h = jnp.maximum(jnp.dot(xs_ref[...], w1buf[slot]) + b1_ref[q], 0.0)
        ys_ref[...] = jnp.dot(h, w2buf[slot]) + b2_ref[q]


def _gmm(ord_t, ebo, first, nact, nord, xs, W1, b1, W2, b2):
    grid_spec = pltpu.PrefetchScalarGridSpec(
        num_scalar_prefetch=5,
        grid=(NT,),
        in_specs=[
            pl.BlockSpec((TILE, D), lambda i, *_: (i, 0)),
            pl.BlockSpec(memory_space=pltpu.MemorySpace.HBM),
            pl.BlockSpec(memory_space=pltpu.MemorySpace.HBM),
            pl.BlockSpec((E, 1, H), lambda i, *_: (0, 0, 0)),
            pl.BlockSpec((E, 1, D), lambda i, *_: (0, 0, 0)),
        ],
        out_specs=pl.BlockSpec((TILE, D), lambda i, *_: (i, 0)),
        scratch_shapes=[
            pltpu.VMEM((_WSLOTS, D, H), jnp.float32),
            pltpu.VMEM((_WSLOTS, H, D), jnp.float32),
            pltpu.SemaphoreType.DMA((_WSLOTS,)),
        ],
    )
    return pl.pallas_call(
        _gmm_body,
        grid_spec=grid_spec,
        out_shape=jax.ShapeDtypeStruct((MPAD, D), jnp.float32),
    )(ord_t, ebo, first, nact, nord, xs, W1, W2, b1.reshape(E, 1, H),
      b2.reshape(E, 1, D))


def kernel(x, Wr, br, W1, b1, W2, b2):
    B, T, _ = x.shape
    xf = x.reshape(M, D)
    (probs, eidx2, pos2, ord2, ebo2, first2, nact2, nord2) = _router(
        xf, Wr, br.reshape(1, E))
    pos = pos2.reshape(M)
    dispatch, combine = _sc_kernels()
    xs = dispatch(xf, pos)
    ys = _gmm(ord2.reshape(64), ebo2.reshape(E), first2.reshape(64),
              nact2.reshape(1), nord2.reshape(1), xs, W1, b1, W2, b2)
    out = combine(ys, pos)
    return (out, probs.reshape(B, T, E), eidx2.reshape(B, T))
